# Initial kernel scaffold; baseline (speedup 1.0000x reference)
import jax
import jax.numpy as jnp
from jax.experimental import pallas as pl


def kernel(x, edge_index, batch, params):
    raise NotImplementedError("write your pallas kernel here")


# speedup vs baseline: 1.6028x; 1.6028x over previous
import functools

import numpy as np
import jax
import jax.numpy as jnp
from jax import lax
from jax.experimental import pallas as pl
from jax.experimental.pallas import tpu as pltpu
from jax.experimental.pallas import tpu_sc as plsc

N = 10000
E = 160000
IN_DIM = 128
N_GRAPHS = 32
NUM_CLASS = 1000
DIMS = [128, 64, 128, 192, 256, 320]
GROUPS = [1, 4, 4, 4, 4]

_NC = 2
_NS = 16
_NW = _NC * _NS
_CHUNK = 128
_E_PAD = 163840
_EPA = _E_PAD // _NW
_NCHA = _EPA // _CHUNK
_EPB = _E_PAD // _NS
_NCHB = _EPB // _CHUNK
_NROWS = 10240
_RPT = _NROWS // _NS
_TROWS = N + 1

_HI = jax.lax.Precision.HIGHEST


def _hswish(x):
    return x * jnp.clip(x + 3.0, 0.0, 6.0) / 6.0


def _bn_exact(x, gamma, beta):
    m = jnp.mean(x, axis=0, keepdims=True)
    v = jnp.mean((x - m) * (x - m), axis=0, keepdims=True)
    return gamma * (x - m) / jnp.sqrt(v + 1e-5) + beta


def _dot_bf(a, b):
    return jnp.dot(a.astype(jnp.bfloat16), b.astype(jnp.bfloat16),
                   preferred_element_type=jnp.float32)


def _make_k1_body(g, ci, co, first):
    def body(*refs):
        if first:
            h_ref, w_ref, b_ref, tp_ref = refs
            h = h_ref[...]
        else:
            tb_ref, ag_ref, w_ref, tp_ref = refs
            h = _hswish(tb_ref[...] + ag_ref[...])
        for k in range(g):
            tp_ref[:, k * co:(k + 1) * co] = _dot_bf(
                h[:, k * ci:(k + 1) * ci], w_ref[k])
        if first:
            tp_ref[...] += b_ref[...]
    return body


def _k1b_body(tp_ref, g_ref, be_ref, t_ref):
    t_ref[...] = _hswish(_bn_exact(tp_ref[...], g_ref[...], be_ref[...]))


def _k2_body(t_ref, w1_ref, b1_ref, w2_ref, b2_ref, esb_ref,
             ebn_ref, tb_ref):
    t = t_ref[...]
    eb1 = _hswish(_dot_bf(t, w1_ref[...]) + b1_ref[...])
    ebn = _hswish(_dot_bf(eb1, w2_ref[...]) + b2_ref[...])
    ebn_ref[...] = ebn
    tb_ref[...] = t + jax.nn.sigmoid(esb_ref[...]) * ebn


def _score_body(d_ref, w_ref, b_ref, s_ref):
    s_ref[...] = jax.nn.sigmoid(_dot_bf(d_ref[...], w_ref[...]) + b_ref[...])


def _head_body(tb_ref, ag_ref, bat_ref, dcw_ref, dcg_ref, dcb_ref,
               cfw_ref, cfg_ref, cfb_ref, fcw_ref, fcb_ref,
               logits_ref, y_ref):
    h = _hswish(tb_ref[...] + ag_ref[...])
    gids = lax.broadcasted_iota(jnp.int32, (N_GRAPHS, 1), 0)
    onehot_t = (bat_ref[...] == gids).astype(jnp.float32)
    sums = jnp.dot(onehot_t, h, preferred_element_type=jnp.float32,
                   precision=_HI)
    cnts = jnp.sum(onehot_t, axis=1, keepdims=True)
    pooled = sums / jnp.maximum(cnts, 1.0)
    y = jnp.concatenate(
        [_dot_bf(pooled[:, k * 80:(k + 1) * 80], dcw_ref[k]) for k in range(4)],
        axis=1)
    y = _hswish(_bn_exact(y, dcg_ref[...], dcb_ref[...]))
    y = jnp.concatenate(
        [_dot_bf(y[:, k * 80:(k + 1) * 80], cfw_ref[k]) for k in range(4)],
        axis=1)
    y = _hswish(_bn_exact(y, cfg_ref[...], cfb_ref[...]))
    y_ref[...] = y
    logits_ref[...] = _dot_bf(y, fcw_ref[...]) + fcb_ref[...]


def _vparams():
    return pltpu.CompilerParams(vmem_limit_bytes=100 * 1024 * 1024)


@functools.lru_cache(maxsize=None)
def _make_gather_diff_kernel(cout):
    mesh = plsc.VectorSubcoreMesh(core_axis_name="c", subcore_axis_name="s")

    @functools.partial(
        pl.kernel, mesh=mesh,
        compiler_params=pltpu.CompilerParams(use_tc_tiling_on_sc=False),
        out_type=jax.ShapeDtypeStruct((_E_PAD, cout), jnp.float32),
        scratch_types=[
            pltpu.VMEM((_EPA,), jnp.int32),
            pltpu.VMEM((_EPA,), jnp.int32),
            pltpu.VMEM((_CHUNK, cout), jnp.float32),
            pltpu.VMEM((_CHUNK, cout), jnp.float32),
            pltpu.SemaphoreType.DMA,
        ],
    )
    def gather_diff(src_h, dst_h, tpad_h, d_h, sidx, didx, rs, rd, sem):
        c = lax.axis_index("c")
        s = lax.axis_index("s")
        w = c * _NS + s
        pltpu.sync_copy(src_h.at[w], sidx)
        pltpu.sync_copy(dst_h.at[w], didx)

        def chunk(j, carry):
            g1 = pltpu.async_copy(
                tpad_h.at[sidx.at[pl.ds(j * _CHUNK, _CHUNK)]], rs, sem)
            g2 = pltpu.async_copy(
                tpad_h.at[didx.at[pl.ds(j * _CHUNK, _CHUNK)]], rd, sem)
            g1.wait()
            g2.wait()

            def erow(e, carry2):
                for k in range(cout // 16):
                    ds = pl.ds(k * 16, 16)
                    rs[e, ds] = rs[e, ds] - rd[e, ds]
                return carry2

            lax.fori_loop(0, _CHUNK, erow, 0, unroll=False)
            pltpu.sync_copy(rs, d_h.at[pl.ds(w * _EPA + j * _CHUNK, _CHUNK)])
            return carry

        lax.fori_loop(0, _NCHA, chunk, 0, unroll=False)

    return gather_diff


@functools.lru_cache(maxsize=None)
def _make_scatter_kernel(quarter):
    mesh = plsc.VectorSubcoreMesh(core_axis_name="c", subcore_axis_name="s")

    @functools.partial(
        pl.kernel, mesh=mesh,
        compiler_params=pltpu.CompilerParams(use_tc_tiling_on_sc=False),
        out_type=jax.ShapeDtypeStruct((4 * N, quarter), jnp.float32),
        scratch_types=[
            pltpu.VMEM((_EPB,), jnp.int32),
            pltpu.VMEM((_NCHB, _CHUNK), jnp.int32),
            pltpu.VMEM((_CHUNK, quarter), jnp.float32),
            pltpu.VMEM((_CHUNK, quarter), jnp.float32),
            pltpu.VMEM_SHARED((_NROWS, quarter), jnp.float32),
            pltpu.SemaphoreType.DMA,
        ],
    )
    def scatter_kernel(src4_h, dsts_h, score_h, ebf_h, zz_h, out_h,
                       sidx, dsts, sc, eb, acc, sem):
        c = lax.axis_index("c")
        s = lax.axis_index("s")
        pltpu.sync_copy(dsts_h.at[s], dsts)
        for p in range(2):
            blk = c * 2 + p
            pltpu.sync_copy(src4_h.at[blk * _NS + s], sidx)
            for z in range(_RPT // _CHUNK):
                pltpu.sync_copy(zz_h,
                                acc.at[pl.ds(s * _RPT + z * _CHUNK, _CHUNK)])
            plsc.subcore_barrier()

            def chunk(j, carry):
                row = s * _EPB + j * _CHUNK
                g1 = pltpu.async_copy(
                    score_h.at[pl.ds(row, _CHUNK),
                               pl.ds(blk * quarter, quarter)], sc, sem)
                g2 = pltpu.async_copy(
                    ebf_h.at[sidx.at[pl.ds(j * _CHUNK, _CHUNK)]], eb, sem)
                g1.wait()
                g2.wait()

                def erow(e, carry2):
                    for k in range(quarter // 16):
                        ds = pl.ds(k * 16, 16)
                        eb[e, ds] = sc[e, ds] * eb[e, ds]
                    return carry2

                lax.fori_loop(0, _CHUNK, erow, 0, unroll=False)
                pltpu.sync_copy(eb, acc.at[dsts.at[j]], add=True)
                return carry

            lax.fori_loop(0, _NCHB, chunk, 0, unroll=False)
            plsc.subcore_barrier()

            @pl.when(s < _NS - 1)
            def _copy_full():
                pltpu.sync_copy(acc.at[pl.ds(s * _RPT, _RPT)],
                                out_h.at[pl.ds(blk * N + s * _RPT, _RPT)])

            @pl.when(s == _NS - 1)
            def _copy_tail():
                pltpu.sync_copy(
                    acc.at[pl.ds((_NS - 1) * _RPT, N - (_NS - 1) * _RPT)],
                    out_h.at[pl.ds(blk * N + (_NS - 1) * _RPT,
                                   N - (_NS - 1) * _RPT)])

            plsc.subcore_barrier()

    return scatter_kernel


def _split_table(a, quarter):
    a = jnp.concatenate([a, jnp.zeros((1, a.shape[1]), a.dtype)], axis=0)
    return a.reshape(_TROWS, 4, quarter).transpose(1, 0, 2).reshape(
        4 * _TROWS, quarter)


def kernel(x, edge_index, batch, params):
    p = params
    src = edge_index[0].astype(jnp.int32)
    dst = edge_index[1].astype(jnp.int32)
    pad = jnp.full((_E_PAD - E,), N, jnp.int32)
    srcp = jnp.concatenate([src, pad])
    dstp = jnp.concatenate([dst, pad])
    srcA = srcp.reshape(_NW, _EPA)
    dstA = dstp.reshape(_NW, _EPA)
    src4 = jnp.stack([srcp + b * _TROWS for b in range(4)]).reshape(
        4 * _NS, _EPB)
    dsts3 = dstp.reshape(_NS, _NCHB, _CHUNK)

    fixed = lambda *shape: pl.BlockSpec(shape, lambda i: (0,) * len(shape))
    blkn = 2000

    h_tbase = None
    h_agg = None
    for i in range(5):
        g = GROUPS[i]
        cin, cout = DIMS[i], DIMS[i + 1]
        ci, co = cin // g, cout // g
        quarter = cout // 4
        perm = np.arange(cout).reshape(g, co).T.reshape(-1)

        if i == 0:
            tp = pl.pallas_call(
                _make_k1_body(g, ci, co, True),
                grid=(N // blkn,),
                in_specs=[pl.BlockSpec((blkn, cin), lambda i: (i, 0)),
                          fixed(g, ci, co), fixed(1, cout)],
                out_specs=pl.BlockSpec((blkn, cout), lambda i: (i, 0)),
                out_shape=jax.ShapeDtypeStruct((N, cout), jnp.float32),
                compiler_params=_vparams(),
            )(x, p['b%d_win' % i], p['b%d_bin' % i].reshape(1, cout))
        else:
            tp = pl.pallas_call(
                _make_k1_body(g, ci, co, False),
                grid=(N // blkn,),
                in_specs=[pl.BlockSpec((blkn, cin), lambda i: (i, 0)),
                          pl.BlockSpec((blkn, cin), lambda i: (i, 0)),
                          fixed(g, ci, co)],
                out_specs=pl.BlockSpec((blkn, cout), lambda i: (i, 0)),
                out_shape=jax.ShapeDtypeStruct((N, cout), jnp.float32),
                compiler_params=_vparams(),
            )(h_tbase, h_agg, p['b%d_win' % i])

        t_pre = pl.pallas_call(
            _k1b_body,
            out_shape=jax.ShapeDtypeStruct((N, cout), jnp.float32),
            compiler_params=_vparams(),
        )(tp, p['b%d_bng' % i].reshape(1, cout), p['b%d_bnb' % i].reshape(1, cout))

        t = t_pre[:, perm] if g > 1 else t_pre

        cq = cout // 4
        ebn, tbase = pl.pallas_call(
            _k2_body,
            grid=(N // blkn,),
            in_specs=[pl.BlockSpec((blkn, cout), lambda i: (i, 0)),
                      fixed(cout, cq), fixed(1, cq),
                      fixed(cq, cout), fixed(1, cout), fixed(1, cout)],
            out_specs=[pl.BlockSpec((blkn, cout), lambda i: (i, 0))] * 2,
            out_shape=[jax.ShapeDtypeStruct((N, cout), jnp.float32)] * 2,
            compiler_params=_vparams(),
        )(t, p['b%d_eb_w1' % i], p['b%d_eb_b1' % i].reshape(1, -1),
          p['b%d_eb_w2' % i], p['b%d_eb_b2' % i].reshape(1, -1),
          p['b%d_es_b' % i].reshape(1, cout))

        tpad = jnp.concatenate([t, jnp.zeros((1, cout), jnp.float32)], axis=0)
        d = _make_gather_diff_kernel(cout)(srcA, dstA, tpad)

        blke = 4096
        score = pl.pallas_call(
            _score_body,
            grid=(_E_PAD // blke,),
            in_specs=[pl.BlockSpec((blke, cout), lambda i: (i, 0)),
                      fixed(cout, cout), fixed(1, cout)],
            out_specs=pl.BlockSpec((blke, cout), lambda i: (i, 0)),
            out_shape=jax.ShapeDtypeStruct((_E_PAD, cout), jnp.float32),
            compiler_params=_vparams(),
        )(d, p['b%d_es_w' % i], p['b%d_es_b' % i].reshape(1, cout))

        ebf = _split_table(ebn, quarter)
        zz = jnp.zeros((_CHUNK, quarter), jnp.float32)
        aggf = _make_scatter_kernel(quarter)(src4, dsts3, score, ebf, zz)
        agg = aggf.reshape(4, N, quarter).transpose(1, 0, 2).reshape(N, cout)

        h_tbase, h_agg = tbase, agg

    logits, y = pl.pallas_call(
        _head_body,
        out_shape=[jax.ShapeDtypeStruct((N_GRAPHS, NUM_CLASS), jnp.float32),
                   jax.ShapeDtypeStruct((N_GRAPHS, 1280), jnp.float32)],
        compiler_params=_vparams(),
    )(h_tbase, h_agg, batch.astype(jnp.int32).reshape(1, N),
      p['dc_w'], p['dc_bng'].reshape(1, -1), p['dc_bnb'].reshape(1, -1),
      p['c2f_w'], p['c2f_bng'].reshape(1, -1), p['c2f_bnb'].reshape(1, -1),
      p['fc_w'], p['fc_b'].reshape(1, -1))
    return logits, y
